# baseline (speedup 1.0000x reference)
import functools

import jax
import jax.numpy as jnp
from jax import lax
from jax.experimental import pallas as pl
from jax.experimental.pallas import tpu as pltpu
from jax.experimental.pallas import tpu_sc as plsc

N_NODES = 10000
N_EDGES = 320000
D_IN = 128
D_HID = 128
N_CLASSES = 40

N_PAD = 10240
NW = 32
CHUNK = 80
N_CHUNKS = 128
EDGES_PER_TILE = N_CHUNKS * CHUNK
E_PAD = NW * EDGES_PER_TILE
ROWS_PER_SUB = N_PAD // 16


def _make_seg_sum(with_deg):
    d = D_HID
    mesh = plsc.VectorSubcoreMesh(core_axis_name="c", subcore_axis_name="s")

    out_type = [
        jax.ShapeDtypeStruct((N_PAD, d), jnp.float32),
        jax.ShapeDtypeStruct((N_PAD, d), jnp.float32),
    ]
    scratch = [
        pltpu.VMEM((2, CHUNK), jnp.int32),
        pltpu.VMEM((2, CHUNK), jnp.int32),
        pltpu.VMEM((2, CHUNK), jnp.int32),
        pltpu.VMEM((3, CHUNK), jnp.int32),
        pltpu.VMEM((CHUNK, d), jnp.float32),
        pltpu.VMEM((CHUNK, d), jnp.float32),
        pltpu.VMEM((CHUNK, d), jnp.float32),
        pltpu.VMEM_SHARED((N_PAD, d), jnp.float32),
    ] + [pltpu.SemaphoreType.DMA] * 9
    if with_deg:
        out_type.append(jax.ShapeDtypeStruct((NW, N_PAD), jnp.float32))
        scratch.append(pltpu.VMEM((N_PAD,), jnp.float32))

    @functools.partial(
        pl.kernel, mesh=mesh, out_type=out_type, scratch_types=scratch,
        compiler_params=pltpu.CompilerParams(needs_layout_passes=False))
    def seg_sum(x_hbm, edges_hbm, zrow_hbm, zdeg_hbm, out0_hbm, out1_hbm,
                *rest):
        if with_deg:
            (outd_hbm, eb0, eb1, eb2, sidx, rows0, rows1, rows2, acc,
             se0, se1, se2, sg0, sg1, sg2, ss0, ss1, ss2, deg_v) = rest
        else:
            (eb0, eb1, eb2, sidx, rows0, rows1, rows2, acc,
             se0, se1, se2, sg0, sg1, sg2, ss0, ss1, ss2) = rest
        EB = [eb0, eb1, eb2]
        ROWS = [rows0, rows1, rows2]
        SE = [se0, se1, se2]
        SG = [sg0, sg1, sg2]
        SS = [ss0, ss1, ss2]
        c = lax.axis_index("c")
        s = lax.axis_index("s")
        wid = c * 16 + s
        row0 = s * ROWS_PER_SUB
        ones = jnp.full((16,), 1.0, jnp.float32)

        pltpu.sync_copy(zrow_hbm, acc.at[pl.ds(row0, ROWS_PER_SUB)])
        if with_deg:
            pltpu.sync_copy(zdeg_hbm, deg_v)
        plsc.subcore_barrier()

        pltpu.sync_copy(edges_hbm.at[wid, 0], eb0)
        pltpu.async_copy(x_hbm.at[eb0.at[0]], rows0, sg0)
        pltpu.async_copy(edges_hbm.at[wid, 1], eb1, se1)

        def step(j, p):
            q = (p + 1) % 3
            r = (p + 2) % 3

            @pl.when(j + 1 < N_CHUNKS)
            def _():
                pltpu.make_async_copy(edges_hbm.at[wid, j + 1], EB[q],
                                      SE[q]).wait()

                @pl.when(j >= 2)
                def _():
                    pltpu.make_async_copy(ROWS[q], acc.at[sidx.at[q]],
                                          SS[q]).wait()

                pltpu.async_copy(x_hbm.at[EB[q].at[0]], ROWS[q], SG[q])

            pltpu.make_async_copy(x_hbm.at[EB[p].at[0]], ROWS[p], SG[p]).wait()
            for k in range(CHUNK // 16):
                idx = EB[p][1, pl.ds(k * 16, 16)]
                sidx[p, pl.ds(k * 16, 16)] = idx
                if with_deg:
                    plsc.addupdate_scatter(deg_v, [idx], ones)
            pltpu.async_copy(ROWS[p], acc.at[sidx.at[p]], SS[p], add=True)

            @pl.when(j + 2 < N_CHUNKS)
            def _():
                pltpu.async_copy(edges_hbm.at[wid, j + 2], EB[r], SE[r])

        def body(j, carry):
            for p in range(3):
                @pl.when(lax.rem(j, 3) == p)
                def _(p=p):
                    step(j, p)

            return carry

        lax.fori_loop(0, N_CHUNKS, body, 0)
        pltpu.make_async_copy(rows0, acc.at[sidx.at[0]], ss0).wait()
        pltpu.make_async_copy(rows1, acc.at[sidx.at[1]], ss1).wait()
        pltpu.make_async_copy(rows2, acc.at[sidx.at[2]], ss2).wait()
        if with_deg:
            pltpu.sync_copy(deg_v, outd_hbm.at[wid])
        plsc.subcore_barrier()

        @pl.when(c == 0)
        def _():
            pltpu.sync_copy(acc.at[pl.ds(row0, ROWS_PER_SUB)],
                            out0_hbm.at[pl.ds(row0, ROWS_PER_SUB)])

        @pl.when(c == 1)
        def _():
            pltpu.sync_copy(acc.at[pl.ds(row0, ROWS_PER_SUB)],
                            out1_hbm.at[pl.ds(row0, ROWS_PER_SUB)])

    return seg_sum


_seg_sum_l1 = _make_seg_sum(with_deg=True)
_seg_sum_l2 = _make_seg_sum(with_deg=False)

_B = 1024


def _tc1_body(p0, p1, degp, xp, w0l, w0r, b0, h1_out, inv_out):
    s = p0[...] + p1[...]
    deg = lax.dot_general(degp[...], jnp.ones((NW, 1), jnp.float32),
                          (((0,), (0,)), ((), ())),
                          preferred_element_type=jnp.float32)
    inv = 1.0 / jnp.maximum(deg, 1.0)
    mean = s * inv
    h = lax.dot_general(mean, w0l[...], (((1,), (1,)), ((), ())),
                        preferred_element_type=jnp.float32)
    h += lax.dot_general(xp[...], w0r[...], (((1,), (1,)), ((), ())),
                         preferred_element_type=jnp.float32)
    h += b0[0:1, :]
    h1_out[...] = jnp.maximum(h, 0.0)
    inv_out[...] = inv


def _tc2_body(q0, q1, inv, h1, w1l, w1r, b1, wlin, blin, out):
    mean = (q0[...] + q1[...]) * inv[...]
    h = lax.dot_general(mean, w1l[...], (((1,), (1,)), ((), ())),
                        preferred_element_type=jnp.float32)
    h += lax.dot_general(h1[...], w1r[...], (((1,), (1,)), ((), ())),
                         preferred_element_type=jnp.float32)
    h += b1[0:1, :]
    h2 = jnp.maximum(h, 0.0)
    z = lax.dot_general(h2, wlin[...], (((1,), (1,)), ((), ())),
                        preferred_element_type=jnp.float32)
    z += blin[0:1, :]
    cols = lax.broadcasted_iota(jnp.int32, z.shape, 1)
    z = jnp.where(cols < N_CLASSES, z, jnp.float32(-1e30))
    m = jnp.max(z, axis=1, keepdims=True)
    lse = jnp.log(jnp.sum(jnp.exp(z - m), axis=1, keepdims=True)) + m
    out[...] = z - lse


def _row_spec(w):
    return pl.BlockSpec((_B, w), lambda i: (i, 0))


def _full_spec(r, w):
    return pl.BlockSpec((r, w), lambda i: (0, 0))


def kernel(x, adj, W0l, b0, W0r, W1l, b1, W1r, Wlin, blin):
    f32 = jnp.float32
    pad = jnp.full((2, E_PAD - N_EDGES), N_NODES, dtype=adj.dtype)
    adj_p = jnp.concatenate([adj, pad], axis=1)
    edges = jnp.stack(
        [adj_p[0].reshape(NW, N_CHUNKS, CHUNK).astype(jnp.int32),
         adj_p[1].reshape(NW, N_CHUNKS, CHUNK).astype(jnp.int32)], axis=2)

    xp = jnp.zeros((N_PAD, D_IN), f32).at[:N_NODES].set(x.astype(f32))
    zrow = jnp.zeros((ROWS_PER_SUB, D_HID), f32)
    zdeg = jnp.zeros((N_PAD,), f32)

    p0, p1, degp = _seg_sum_l1(xp, edges, zrow, zdeg)

    b0b = jnp.broadcast_to(b0.astype(f32)[None, :], (8, D_HID))
    b1b = jnp.broadcast_to(b1.astype(f32)[None, :], (8, D_HID))
    wlin_p = jnp.zeros((D_HID, D_HID), f32).at[:N_CLASSES, :].set(Wlin.astype(f32))
    blin_p = jnp.zeros((D_HID,), f32).at[:N_CLASSES].set(blin.astype(f32))
    blin_b = jnp.broadcast_to(blin_p[None, :], (8, D_HID))

    h1, inv_deg = pl.pallas_call(
        _tc1_body,
        grid=(N_PAD // _B,),
        in_specs=[
            _row_spec(D_HID),
            _row_spec(D_HID),
            pl.BlockSpec((NW, _B), lambda i: (0, i)),
            _row_spec(D_IN),
            _full_spec(D_HID, D_IN),
            _full_spec(D_HID, D_IN),
            _full_spec(8, D_HID),
        ],
        out_specs=[_row_spec(D_HID), _row_spec(1)],
        out_shape=[
            jax.ShapeDtypeStruct((N_PAD, D_HID), f32),
            jax.ShapeDtypeStruct((N_PAD, 1), f32),
        ],
    )(p0, p1, degp, xp, W0l.astype(f32), W0r.astype(f32), b0b)

    q0, q1 = _seg_sum_l2(h1, edges, zrow, zdeg)

    out_full = pl.pallas_call(
        _tc2_body,
        grid=(N_PAD // _B,),
        in_specs=[
            _row_spec(D_HID),
            _row_spec(D_HID),
            _row_spec(1),
            _row_spec(D_HID),
            _full_spec(D_HID, D_HID),
            _full_spec(D_HID, D_HID),
            _full_spec(8, D_HID),
            _full_spec(D_HID, D_HID),
            _full_spec(8, D_HID),
        ],
        out_specs=pl.BlockSpec((_B, D_HID), lambda i: (i, 0)),
        out_shape=jax.ShapeDtypeStruct((N_PAD, D_HID), f32),
    )(q0, q1, inv_deg, h1, W1l.astype(f32), W1r.astype(f32), b1b, wlin_p,
      blin_b)

    return out_full[:N_NODES, :N_CLASSES]


# speedup vs baseline: 2.4957x; 2.4957x over previous
import functools

import jax
import jax.numpy as jnp
from jax import lax
from jax.experimental import pallas as pl
from jax.experimental.pallas import tpu as pltpu
from jax.experimental.pallas import tpu_sc as plsc

N_NODES = 10000
N_EDGES = 320000
D_IN = 128
D_HID = 128
N_CLASSES = 40

N_PAD = 10240
NW = 32
EDGES_PER_TILE = N_EDGES // NW
CHUNK = 80
N_CHUNKS = EDGES_PER_TILE // CHUNK
ROWS_PER_SUB = N_PAD // 16


def _make_seg_sum(with_deg):
    d = D_HID
    mesh = plsc.VectorSubcoreMesh(core_axis_name="c", subcore_axis_name="s")

    out_type = [
        jax.ShapeDtypeStruct((N_PAD, d), jnp.float32),
        jax.ShapeDtypeStruct((N_PAD, d), jnp.float32),
    ]
    scratch = [
        pltpu.VMEM((2, CHUNK), jnp.int32),
        pltpu.VMEM((2, CHUNK), jnp.int32),
        pltpu.VMEM((CHUNK, d), jnp.float32),
        pltpu.VMEM((CHUNK, d), jnp.float32),
        pltpu.VMEM_SHARED((N_PAD, d), jnp.float32),
        pltpu.SemaphoreType.DMA,
        pltpu.SemaphoreType.DMA,
        pltpu.SemaphoreType.DMA,
        pltpu.SemaphoreType.DMA,
    ]
    if with_deg:
        out_type.append(jax.ShapeDtypeStruct((NW, N_PAD), jnp.float32))
        scratch.append(pltpu.VMEM((N_PAD,), jnp.float32))

    @functools.partial(
        pl.kernel, mesh=mesh, out_type=out_type, scratch_types=scratch,
        compiler_params=pltpu.CompilerParams(needs_layout_passes=False))
    def seg_sum(x_hbm, edges_hbm, zrow_hbm, zdeg_hbm, out0_hbm, out1_hbm,
                *rest):
        if with_deg:
            outd_hbm, eb0, eb1, rows0, rows1, acc, se0, se1, sg0, sg1, deg_v = rest
        else:
            eb0, eb1, rows0, rows1, acc, se0, se1, sg0, sg1 = rest
        c = lax.axis_index("c")
        s = lax.axis_index("s")
        wid = c * 16 + s
        row0 = s * ROWS_PER_SUB
        ones = jnp.full((16,), 1.0, jnp.float32)

        pltpu.sync_copy(zrow_hbm, acc.at[pl.ds(row0, ROWS_PER_SUB)])
        if with_deg:
            pltpu.sync_copy(zdeg_hbm, deg_v)
        plsc.subcore_barrier()

        pltpu.sync_copy(edges_hbm.at[wid, 0], eb0)
        pltpu.async_copy(x_hbm.at[eb0.at[0]], rows0, sg0)
        pltpu.async_copy(edges_hbm.at[wid, 1], eb1, se1)

        def step(j, eb, rows, se, sg, eb_o, rows_o, se_o, sg_o):
            @pl.when(j + 1 < N_CHUNKS)
            def _():
                pltpu.make_async_copy(edges_hbm.at[wid, j + 1], eb_o,
                                      se_o).wait()
                pltpu.async_copy(x_hbm.at[eb_o.at[0]], rows_o, sg_o)

            pltpu.make_async_copy(x_hbm.at[eb.at[0]], rows, sg).wait()
            pltpu.sync_copy(rows, acc.at[eb.at[1]], add=True)

            if with_deg:
                for k in range(CHUNK // 16):
                    idx = eb[1, pl.ds(k * 16, 16)]
                    plsc.addupdate_scatter(deg_v, [idx], ones)

            @pl.when(j + 2 < N_CHUNKS)
            def _():
                pltpu.async_copy(edges_hbm.at[wid, j + 2], eb, se)

        def body(j, carry):
            @pl.when(lax.rem(j, 2) == 0)
            def _():
                step(j, eb0, rows0, se0, sg0, eb1, rows1, se1, sg1)

            @pl.when(lax.rem(j, 2) == 1)
            def _():
                step(j, eb1, rows1, se1, sg1, eb0, rows0, se0, sg0)

            return carry

        lax.fori_loop(0, N_CHUNKS, body, 0)
        if with_deg:
            pltpu.sync_copy(deg_v, outd_hbm.at[wid])
        plsc.subcore_barrier()

        @pl.when(c == 0)
        def _():
            pltpu.sync_copy(acc.at[pl.ds(row0, ROWS_PER_SUB)],
                            out0_hbm.at[pl.ds(row0, ROWS_PER_SUB)])

        @pl.when(c == 1)
        def _():
            pltpu.sync_copy(acc.at[pl.ds(row0, ROWS_PER_SUB)],
                            out1_hbm.at[pl.ds(row0, ROWS_PER_SUB)])

    return seg_sum


_seg_sum_l1 = _make_seg_sum(with_deg=True)
_seg_sum_l2 = _make_seg_sum(with_deg=False)

_B = 1024


def _tc_root_body(xp, w, b, out):
    h = lax.dot_general(xp[...], w[...], (((1,), (1,)), ((), ())),
                        preferred_element_type=jnp.float32)
    out[...] = h + b[0:1, :]


def _tc1_body(p0, p1, degp, xr, w0l, h1_out, inv_out):
    s = p0[...] + p1[...]
    deg = lax.dot_general(degp[...], jnp.ones((NW, 1), jnp.float32),
                          (((0,), (0,)), ((), ())),
                          preferred_element_type=jnp.float32)
    inv = 1.0 / jnp.maximum(deg, 1.0)
    mean = s * inv
    h = lax.dot_general(mean, w0l[...], (((1,), (1,)), ((), ())),
                        preferred_element_type=jnp.float32)
    h1_out[...] = jnp.maximum(h + xr[...], 0.0)
    inv_out[...] = inv


def _tc2_body(q0, q1, inv, r2, w1l, wlin, blin, out):
    mean = (q0[...] + q1[...]) * inv[...]
    h = lax.dot_general(mean, w1l[...], (((1,), (1,)), ((), ())),
                        preferred_element_type=jnp.float32)
    h2 = jnp.maximum(h + r2[...], 0.0)
    z = lax.dot_general(h2, wlin[...], (((1,), (1,)), ((), ())),
                        preferred_element_type=jnp.float32)
    z += blin[0:1, :]
    cols = lax.broadcasted_iota(jnp.int32, z.shape, 1)
    z = jnp.where(cols < N_CLASSES, z, jnp.float32(-1e30))
    m = jnp.max(z, axis=1, keepdims=True)
    lse = jnp.log(jnp.sum(jnp.exp(z - m), axis=1, keepdims=True)) + m
    out[...] = z - lse


def _row_spec(w):
    return pl.BlockSpec((_B, w), lambda i: (i, 0))


def _full_spec(r, w):
    return pl.BlockSpec((r, w), lambda i: (0, 0))


def _tc_root(xp, w, bb):
    return pl.pallas_call(
        _tc_root_body,
        grid=(N_PAD // _B,),
        in_specs=[
            _row_spec(D_HID),
            _full_spec(D_HID, D_HID),
            _full_spec(8, D_HID),
        ],
        out_specs=_row_spec(D_HID),
        out_shape=jax.ShapeDtypeStruct((N_PAD, D_HID), jnp.float32),
    )(xp, w, bb)


def kernel(x, adj, W0l, b0, W0r, W1l, b1, W1r, Wlin, blin):
    f32 = jnp.float32
    edges = jnp.stack(
        [adj[0].reshape(NW, N_CHUNKS, CHUNK).astype(jnp.int32),
         adj[1].reshape(NW, N_CHUNKS, CHUNK).astype(jnp.int32)], axis=2)

    xp = jnp.zeros((N_PAD, D_IN), f32).at[:N_NODES].set(x.astype(f32))
    zrow = jnp.zeros((ROWS_PER_SUB, D_HID), f32)
    zdeg = jnp.zeros((N_PAD,), f32)

    b0b = jnp.broadcast_to(b0.astype(f32)[None, :], (8, D_HID))
    b1b = jnp.broadcast_to(b1.astype(f32)[None, :], (8, D_HID))
    wlin_p = jnp.zeros((D_HID, D_HID), f32).at[:N_CLASSES, :].set(Wlin.astype(f32))
    blin_p = jnp.zeros((D_HID,), f32).at[:N_CLASSES].set(blin.astype(f32))
    blin_b = jnp.broadcast_to(blin_p[None, :], (8, D_HID))

    p0, p1, degp = _seg_sum_l1(xp, edges, zrow, zdeg)
    xr = _tc_root(xp, W0r.astype(f32), b0b)

    h1, inv_deg = pl.pallas_call(
        _tc1_body,
        grid=(N_PAD // _B,),
        in_specs=[
            _row_spec(D_HID),
            _row_spec(D_HID),
            pl.BlockSpec((NW, _B), lambda i: (0, i)),
            _row_spec(D_HID),
            _full_spec(D_HID, D_IN),
        ],
        out_specs=[_row_spec(D_HID), _row_spec(1)],
        out_shape=[
            jax.ShapeDtypeStruct((N_PAD, D_HID), f32),
            jax.ShapeDtypeStruct((N_PAD, 1), f32),
        ],
    )(p0, p1, degp, xr, W0l.astype(f32))

    q0, q1 = _seg_sum_l2(h1, edges, zrow, zdeg)
    r2 = _tc_root(h1, W1r.astype(f32), b1b)

    out_full = pl.pallas_call(
        _tc2_body,
        grid=(N_PAD // _B,),
        in_specs=[
            _row_spec(D_HID),
            _row_spec(D_HID),
            _row_spec(1),
            _row_spec(D_HID),
            _full_spec(D_HID, D_HID),
            _full_spec(D_HID, D_HID),
            _full_spec(8, D_HID),
        ],
        out_specs=pl.BlockSpec((_B, D_HID), lambda i: (i, 0)),
        out_shape=jax.ShapeDtypeStruct((N_PAD, D_HID), f32),
    )(q0, q1, inv_deg, r2, W1l.astype(f32), wlin_p, blin_b)

    return out_full[:N_NODES, :N_CLASSES]


# speedup vs baseline: 2.5093x; 1.0054x over previous
import functools

import jax
import jax.numpy as jnp
from jax import lax
from jax.experimental import pallas as pl
from jax.experimental.pallas import tpu as pltpu
from jax.experimental.pallas import tpu_sc as plsc

N_NODES = 10000
N_EDGES = 320000
D_IN = 128
D_HID = 128
N_CLASSES = 40

N_PAD = 10240
NW = 32
EDGES_PER_TILE = N_EDGES // NW
CHUNK = 80
N_CHUNKS = EDGES_PER_TILE // CHUNK
ROWS_PER_SUB = N_PAD // 16


def _make_seg_sum(with_deg):
    d = D_HID
    mesh = plsc.VectorSubcoreMesh(core_axis_name="c", subcore_axis_name="s")

    out_type = [
        jax.ShapeDtypeStruct((N_PAD, d), jnp.float32),
        jax.ShapeDtypeStruct((N_PAD, d), jnp.float32),
    ]
    scratch = [
        pltpu.VMEM((2, CHUNK), jnp.int32),
        pltpu.VMEM((2, CHUNK), jnp.int32),
        pltpu.VMEM((CHUNK, d), jnp.float32),
        pltpu.VMEM((CHUNK, d), jnp.float32),
        pltpu.VMEM_SHARED((N_PAD, d), jnp.float32),
        pltpu.SemaphoreType.DMA,
        pltpu.SemaphoreType.DMA,
        pltpu.SemaphoreType.DMA,
        pltpu.SemaphoreType.DMA,
    ]
    if with_deg:
        out_type.append(jax.ShapeDtypeStruct((NW, N_PAD), jnp.float32))
        scratch.append(pltpu.VMEM((N_PAD,), jnp.float32))

    @functools.partial(
        pl.kernel, mesh=mesh, out_type=out_type, scratch_types=scratch,
        compiler_params=pltpu.CompilerParams(needs_layout_passes=False))
    def seg_sum(x_hbm, edges_hbm, zrow_hbm, zdeg_hbm, out0_hbm, out1_hbm,
                *rest):
        if with_deg:
            outd_hbm, eb0, eb1, rows0, rows1, acc, se0, se1, sg0, sg1, deg_v = rest
        else:
            eb0, eb1, rows0, rows1, acc, se0, se1, sg0, sg1 = rest
        c = lax.axis_index("c")
        s = lax.axis_index("s")
        wid = c * 16 + s
        row0 = s * ROWS_PER_SUB
        ones = jnp.full((16,), 1.0, jnp.float32)

        pltpu.sync_copy(zrow_hbm, acc.at[pl.ds(row0, ROWS_PER_SUB)])
        if with_deg:
            pltpu.sync_copy(zdeg_hbm, deg_v)
        plsc.subcore_barrier()

        pltpu.sync_copy(edges_hbm.at[wid, 0], eb0)
        pltpu.async_copy(x_hbm.at[eb0.at[0]], rows0, sg0)
        pltpu.async_copy(edges_hbm.at[wid, 1], eb1, se1)

        def step(j, eb, rows, se, sg, eb_o, rows_o, se_o, sg_o):
            @pl.when(j + 1 < N_CHUNKS)
            def _():
                pltpu.make_async_copy(edges_hbm.at[wid, j + 1], eb_o,
                                      se_o).wait()
                pltpu.async_copy(x_hbm.at[eb_o.at[0]], rows_o, sg_o)

            pltpu.make_async_copy(x_hbm.at[eb.at[0]], rows, sg).wait()
            pltpu.sync_copy(rows, acc.at[eb.at[1]], add=True)

            if with_deg:
                for k in range(CHUNK // 16):
                    idx = eb[1, pl.ds(k * 16, 16)]
                    plsc.addupdate_scatter(deg_v, [idx], ones)

            @pl.when(j + 2 < N_CHUNKS)
            def _():
                pltpu.async_copy(edges_hbm.at[wid, j + 2], eb, se)

        def body(j, carry):
            @pl.when(lax.rem(j, 2) == 0)
            def _():
                step(j, eb0, rows0, se0, sg0, eb1, rows1, se1, sg1)

            @pl.when(lax.rem(j, 2) == 1)
            def _():
                step(j, eb1, rows1, se1, sg1, eb0, rows0, se0, sg0)

            return carry

        lax.fori_loop(0, N_CHUNKS, body, 0)
        if with_deg:
            pltpu.sync_copy(deg_v, outd_hbm.at[wid])
        plsc.subcore_barrier()

        @pl.when(c == 0)
        def _():
            pltpu.sync_copy(acc.at[pl.ds(row0, ROWS_PER_SUB)],
                            out0_hbm.at[pl.ds(row0, ROWS_PER_SUB)])

        @pl.when(c == 1)
        def _():
            pltpu.sync_copy(acc.at[pl.ds(row0, ROWS_PER_SUB)],
                            out1_hbm.at[pl.ds(row0, ROWS_PER_SUB)])

    return seg_sum


_seg_sum_l1 = _make_seg_sum(with_deg=True)
_seg_sum_l2 = _make_seg_sum(with_deg=False)

_B = 1024


def _tc1_body(p0, p1, degp, xp, w0l, w0r, b0, h1_out, inv_out):
    s = p0[...] + p1[...]
    deg = lax.dot_general(degp[...], jnp.ones((NW, 1), jnp.float32),
                          (((0,), (0,)), ((), ())),
                          preferred_element_type=jnp.float32)
    inv = 1.0 / jnp.maximum(deg, 1.0)
    mean = s * inv
    h = lax.dot_general(mean, w0l[...], (((1,), (1,)), ((), ())),
                        preferred_element_type=jnp.float32)
    h += lax.dot_general(xp[...], w0r[...], (((1,), (1,)), ((), ())),
                         preferred_element_type=jnp.float32)
    h += b0[0:1, :]
    h1_out[...] = jnp.maximum(h, 0.0)
    inv_out[...] = inv


def _tc2_body(q0, q1, inv, h1, w1l, w1r, b1, wlin, blin, out):
    mean = (q0[...] + q1[...]) * inv[...]
    h = lax.dot_general(mean, w1l[...], (((1,), (1,)), ((), ())),
                        preferred_element_type=jnp.float32)
    h += lax.dot_general(h1[...], w1r[...], (((1,), (1,)), ((), ())),
                         preferred_element_type=jnp.float32)
    h += b1[0:1, :]
    h2 = jnp.maximum(h, 0.0)
    z = lax.dot_general(h2, wlin[...], (((1,), (1,)), ((), ())),
                        preferred_element_type=jnp.float32)
    z += blin[0:1, :]
    cols = lax.broadcasted_iota(jnp.int32, z.shape, 1)
    z = jnp.where(cols < N_CLASSES, z, jnp.float32(-1e30))
    m = jnp.max(z, axis=1, keepdims=True)
    lse = jnp.log(jnp.sum(jnp.exp(z - m), axis=1, keepdims=True)) + m
    out[...] = z - lse


def _row_spec(w):
    return pl.BlockSpec((_B, w), lambda i: (i, 0))


def _full_spec(r, w):
    return pl.BlockSpec((r, w), lambda i: (0, 0))


def kernel(x, adj, W0l, b0, W0r, W1l, b1, W1r, Wlin, blin):
    f32 = jnp.float32
    edges = jnp.stack(
        [adj[0].reshape(NW, N_CHUNKS, CHUNK).astype(jnp.int32),
         adj[1].reshape(NW, N_CHUNKS, CHUNK).astype(jnp.int32)], axis=2)

    xp = jnp.zeros((N_PAD, D_IN), f32).at[:N_NODES].set(x.astype(f32))
    zrow = jnp.zeros((ROWS_PER_SUB, D_HID), f32)
    zdeg = jnp.zeros((N_PAD,), f32)

    b0b = jnp.broadcast_to(b0.astype(f32)[None, :], (8, D_HID))
    b1b = jnp.broadcast_to(b1.astype(f32)[None, :], (8, D_HID))
    wlin_p = jnp.zeros((D_HID, D_HID), f32).at[:N_CLASSES, :].set(Wlin.astype(f32))
    blin_p = jnp.zeros((D_HID,), f32).at[:N_CLASSES].set(blin.astype(f32))
    blin_b = jnp.broadcast_to(blin_p[None, :], (8, D_HID))

    p0, p1, degp = _seg_sum_l1(xp, edges, zrow, zdeg)

    h1, inv_deg = pl.pallas_call(
        _tc1_body,
        grid=(N_PAD // _B,),
        in_specs=[
            _row_spec(D_HID),
            _row_spec(D_HID),
            pl.BlockSpec((NW, _B), lambda i: (0, i)),
            _row_spec(D_IN),
            _full_spec(D_HID, D_IN),
            _full_spec(D_HID, D_IN),
            _full_spec(8, D_HID),
        ],
        out_specs=[_row_spec(D_HID), _row_spec(1)],
        out_shape=[
            jax.ShapeDtypeStruct((N_PAD, D_HID), f32),
            jax.ShapeDtypeStruct((N_PAD, 1), f32),
        ],
    )(p0, p1, degp, xp, W0l.astype(f32), W0r.astype(f32), b0b)

    q0, q1 = _seg_sum_l2(h1, edges, zrow, zdeg)

    out_full = pl.pallas_call(
        _tc2_body,
        grid=(N_PAD // _B,),
        in_specs=[
            _row_spec(D_HID),
            _row_spec(D_HID),
            _row_spec(1),
            _row_spec(D_HID),
            _full_spec(D_HID, D_HID),
            _full_spec(D_HID, D_HID),
            _full_spec(8, D_HID),
            _full_spec(D_HID, D_HID),
            _full_spec(8, D_HID),
        ],
        out_specs=pl.BlockSpec((_B, D_HID), lambda i: (i, 0)),
        out_shape=jax.ShapeDtypeStruct((N_PAD, D_HID), f32),
    )(q0, q1, inv_deg, h1, W1l.astype(f32), W1r.astype(f32), b1b, wlin_p,
      blin_b)

    return out_full[:N_NODES, :N_CLASSES]
